# baseline (speedup 1.0000x reference)
import functools

import jax
import jax.numpy as jnp
from jax.experimental import pallas as pl
from jax.experimental.pallas import tpu as pltpu

_K = 256
_ROWS_PER_BLOCK = 32


def _key_to_f32(u):
    s = jnp.where(u >= jnp.uint32(0x80000000), u ^ jnp.uint32(0x80000000), ~u)
    return jax.lax.bitcast_convert_type(s, jnp.float32)


def _f32_to_key(x):
    s = jax.lax.bitcast_convert_type(x, jnp.uint32)
    return jnp.where(s >= jnp.uint32(0x80000000), ~s, s | jnp.uint32(0x80000000))


def _topk_mask_kernel(x_ref, out_ref, yb_ref, k):
    rows = x_ref.shape[0]
    cols = x_ref.shape[1]
    n_sl = cols // 256

    accs = [x_ref[:, 256 * i:256 * (i + 1)] for i in range(4)]
    for i in range(4, n_sl):
        accs[i % 4] = jnp.maximum(accs[i % 4], x_ref[:, 256 * i:256 * (i + 1)])
    g = jnp.maximum(jnp.maximum(accs[0], accs[1]),
                    jnp.maximum(accs[2], accs[3]))
    zero = jnp.float32(0.0)
    lo0 = _f32_to_key(jnp.maximum(jnp.min(g, axis=1, keepdims=True), zero))
    hi0 = _f32_to_key(jnp.maximum(jnp.max(g, axis=1, keepdims=True), zero))
    hi0 = hi0 + jnp.uint32(1)

    s = jax.lax.bitcast_convert_type(jnp.maximum(x_ref[...], zero), jnp.uint32)
    yb_ref[...] = jax.lax.bitcast_convert_type(
        (s >> jnp.uint32(16)).astype(jnp.uint16), jnp.bfloat16)

    def cond_a(carry):
        lo, hi = carry
        return jnp.any((hi >> jnp.uint32(16)) - (lo >> jnp.uint32(16))
                       > jnp.uint32(1))

    def body_a(carry):
        lo, hi = carry
        blo = lo >> jnp.uint32(16)
        midb = blo + (((hi >> jnp.uint32(16)) - blo) >> jnp.uint32(1))
        mid = midb << jnp.uint32(16)
        tb = jax.lax.bitcast_convert_type(
            (midb ^ jnp.uint32(0x8000)).astype(jnp.uint16), jnp.bfloat16)
        cnt = jnp.sum((yb_ref[...] >= tb).astype(jnp.float32), axis=1,
                      keepdims=True)
        ge = cnt >= k
        return jnp.where(ge, mid, lo), jnp.where(ge, hi, mid)

    lo, hi = jax.lax.while_loop(cond_a, body_a, (lo0, hi0))

    def cond_b(carry):
        lo, hi = carry
        return jnp.any((hi - lo) > jnp.uint32(1))

    def body_b(carry):
        lo, hi = carry
        mid = lo + ((hi - lo) >> jnp.uint32(1))
        t = _key_to_f32(mid)
        cnt = jnp.sum((x_ref[...] >= t).astype(jnp.float32), axis=1,
                      keepdims=True)
        ge = cnt >= k
        return jnp.where(ge, mid, lo), jnp.where(ge, hi, mid)

    lo, _ = jax.lax.while_loop(cond_b, body_b, (lo, hi))
    t = _key_to_f32(lo)
    x = x_ref[...]
    out_ref[...] = jnp.where((x >= t) & (x > 0.0), x, 0.0)


def kernel(x):
    n_rows, n_cols = x.shape
    r = _ROWS_PER_BLOCK
    grid = (n_rows // r,)
    return pl.pallas_call(
        functools.partial(_topk_mask_kernel, k=_K),
        grid=grid,
        in_specs=[pl.BlockSpec((r, n_cols), lambda i: (i, 0))],
        out_specs=pl.BlockSpec((r, n_cols), lambda i: (i, 0)),
        out_shape=jax.ShapeDtypeStruct(x.shape, x.dtype),
        scratch_shapes=[pltpu.VMEM((r, n_cols), jnp.bfloat16)],
    )(x)


# speedup vs baseline: 1.0435x; 1.0435x over previous
import functools

import jax
import jax.numpy as jnp
from jax.experimental import pallas as pl
from jax.experimental.pallas import tpu as pltpu

_K = 256
_ROWS_PER_BLOCK = 32


def _key_to_f32(u):
    s = jnp.where(u >= jnp.uint32(0x80000000), u ^ jnp.uint32(0x80000000), ~u)
    return jax.lax.bitcast_convert_type(s, jnp.float32)


def _f32_to_key(x):
    s = jax.lax.bitcast_convert_type(x, jnp.uint32)
    return jnp.where(s >= jnp.uint32(0x80000000), ~s, s | jnp.uint32(0x80000000))


def _topk_mask_kernel(x_ref, out_ref, k):
    rows = x_ref.shape[0]
    cols = x_ref.shape[1]
    half = rows // 2
    n_sl = cols // 256

    accs = [x_ref[:, 256 * i:256 * (i + 1)] for i in range(4)]
    for i in range(4, n_sl):
        accs[i % 4] = jnp.maximum(accs[i % 4], x_ref[:, 256 * i:256 * (i + 1)])
    g = jnp.maximum(jnp.maximum(accs[0], accs[1]),
                    jnp.maximum(accs[2], accs[3]))
    lo0 = _f32_to_key(jnp.min(g, axis=1, keepdims=True))
    hi0 = _f32_to_key(jnp.max(g, axis=1, keepdims=True)) + jnp.uint32(1)

    def count(t):
        c0 = jnp.sum((x_ref[0:half, :] >= t[0:half]).astype(jnp.float32),
                     axis=1, keepdims=True)
        c1 = jnp.sum((x_ref[half:rows, :] >= t[half:rows]).astype(jnp.float32),
                     axis=1, keepdims=True)
        return jnp.concatenate([c0, c1], axis=0)

    def step(carry):
        lo, hi = carry
        mid = lo + ((hi - lo) >> jnp.uint32(1))
        cnt = count(_key_to_f32(mid))
        ge = cnt >= k
        return jnp.where(ge, mid, lo), jnp.where(ge, hi, mid)

    def cond(carry):
        lo, hi = carry
        return jnp.any((hi - lo) > jnp.uint32(1))

    def body(carry):
        return step(step(carry))

    lo, _ = jax.lax.while_loop(cond, body, (lo0, hi0))
    t = _key_to_f32(lo)
    x = x_ref[...]
    out_ref[...] = jnp.where((x >= t) & (x > 0.0), x, 0.0)


def kernel(x):
    n_rows, n_cols = x.shape
    r = _ROWS_PER_BLOCK
    grid = (n_rows // r,)
    return pl.pallas_call(
        functools.partial(_topk_mask_kernel, k=_K),
        grid=grid,
        in_specs=[pl.BlockSpec((r, n_cols), lambda i: (i, 0))],
        out_specs=pl.BlockSpec((r, n_cols), lambda i: (i, 0)),
        out_shape=jax.ShapeDtypeStruct(x.shape, x.dtype),
    )(x)
